# Initial kernel scaffold; baseline (speedup 1.0000x reference)
import functools

import jax
import jax.numpy as jnp
from jax import lax
from jax.experimental import pallas as pl
from jax.experimental.pallas import tpu as pltpu
from jax.experimental.pallas import tpu_sc as plsc

N_NODES = 10000
N_EDGES = 320000
OUT_CH = 128

NC = 2
NS = 16
L = 16
NW = NC * NS

E_PER_TILE = N_EDGES // NW
CHUNK = 128
FULL_CHUNKS = E_PER_TILE // CHUNK
TAIL = E_PER_TILE - FULL_CHUNKS * CHUNK
ROWS_PER_TILE = N_NODES // NS


def _prep_body(x_ref, w_ref, asrc_ref, adst_ref, h_ref, as_ref, ad_ref,
               gm_ref):
    h = jnp.dot(x_ref[...], w_ref[...], preferred_element_type=jnp.float32)
    h_ref[...] = h
    a_s = jnp.sum(h * asrc_ref[...], axis=-1, keepdims=True)
    a_d = jnp.sum(h * adst_ref[...], axis=-1, keepdims=True)
    as_ref[...] = a_s
    ad_ref[...] = a_d
    g = jnp.max(a_s) + jnp.max(a_d)
    gm_ref[0, 0] = jnp.where(g >= 0.0, g, 0.2 * g)


def _prep_call(x, W, att_src, att_dst):
    return pl.pallas_call(
        _prep_body,
        out_shape=[
            jax.ShapeDtypeStruct((N_NODES, OUT_CH), jnp.float32),
            jax.ShapeDtypeStruct((N_NODES, 1), jnp.float32),
            jax.ShapeDtypeStruct((N_NODES, 1), jnp.float32),
            jax.ShapeDtypeStruct((1, 1), jnp.float32),
        ],
    )(x, W, att_src, att_dst)


_mesh = plsc.VectorSubcoreMesh(core_axis_name="c", subcore_axis_name="s",
                               num_cores=NC, num_subcores=NS)


@functools.partial(
    pl.kernel,
    out_type=[
        jax.ShapeDtypeStruct((NC, N_NODES, OUT_CH), jnp.float32),
        jax.ShapeDtypeStruct((NC, N_NODES), jnp.float32),
    ],
    mesh=_mesh,
    scratch_types=[
        pltpu.VMEM((N_NODES,), jnp.float32),
        pltpu.VMEM((N_NODES,), jnp.float32),
        pltpu.VMEM((L,), jnp.float32),
        pltpu.VMEM((CHUNK,), jnp.int32),
        pltpu.VMEM((CHUNK,), jnp.int32),
        pltpu.VMEM((CHUNK, OUT_CH), jnp.float32),
        pltpu.VMEM((CHUNK,), jnp.float32),
        pltpu.VMEM((TAIL,), jnp.int32),
        pltpu.VMEM((TAIL,), jnp.int32),
        pltpu.VMEM((TAIL, OUT_CH), jnp.float32),
        pltpu.VMEM((TAIL,), jnp.float32),
        pltpu.VMEM((1024,), jnp.float32),
        pltpu.VMEM_SHARED((N_NODES, OUT_CH), jnp.float32),
        pltpu.VMEM_SHARED((N_NODES,), jnp.float32),
        pltpu.SemaphoreType.DMA,
    ],
)
def _sc_edges(src_hbm, dst_hbm, as_hbm, ad_hbm, gm_hbm, h_hbm,
              num_hbm, den_hbm,
              as_l, ad_l, gm_l, src_c, dst_c, rows, w_c,
              src_t, dst_t, rows_t, w_t, zb1,
              num_sh, den_sh, sem):
    cid = lax.axis_index("c")
    sid = lax.axis_index("s")
    wid = cid * NS + sid

    pltpu.sync_copy(as_hbm, as_l)
    pltpu.sync_copy(ad_hbm, ad_l)
    pltpu.sync_copy(gm_hbm, gm_l)
    gm_vec = gm_l[...]

    zero16 = jnp.zeros((L,), jnp.float32)

    def _z1(i, _):
        zb1[pl.ds(i * L, L)] = zero16
        return 0
    lax.fori_loop(0, 1024 // L, _z1, 0)

    def _zrow(i, _):
        for j in range(OUT_CH // L):
            rows[i, pl.ds(j * L, L)] = zero16
        return 0
    lax.fori_loop(0, CHUNK, _zrow, 0)

    base_r = sid * ROWS_PER_TILE
    for off, nrows in ((0, 128), (128, 128), (256, 128), (384, 128),
                       (512, 113)):
        pltpu.sync_copy(rows.at[pl.ds(0, nrows)],
                        num_sh.at[pl.ds(base_r + off, nrows)])

    @pl.when(sid < 10)
    def _zden():
        pltpu.sync_copy(zb1.at[pl.ds(0, 1000)],
                        den_sh.at[pl.ds(sid * 1000, 1000)])

    plsc.subcore_barrier()

    tile_base = wid * E_PER_TILE

    def _do_chunk(base, k, s_ref, d_ref, r_ref, wref):
        pltpu.sync_copy(src_hbm.at[pl.ds(base, k)], s_ref)
        pltpu.sync_copy(dst_hbm.at[pl.ds(base, k)], d_ref)
        pltpu.async_copy(h_hbm.at[s_ref], r_ref, sem).wait()
        for v in range(k // L):
            si = s_ref[pl.ds(v * L, L)]
            di = d_ref[pl.ds(v * L, L)]
            a_s = plsc.load_gather(as_l, [si])
            a_d = plsc.load_gather(ad_l, [di])
            e = a_s + a_d
            e = jnp.where(e >= 0.0, e, 0.2 * e)
            wref[pl.ds(v * L, L)] = jnp.exp(e - gm_vec)

        def _scale(i, _):
            ws = wref[i]
            for j in range(OUT_CH // L):
                r_ref[i, pl.ds(j * L, L)] = r_ref[i, pl.ds(j * L, L)] * ws
            return 0
        lax.fori_loop(0, k, _scale, 0)
        pltpu.sync_copy(r_ref, num_sh.at[d_ref], add=True)
        pltpu.sync_copy(wref, den_sh.at[d_ref], add=True)

    def _chunk_loop(c, _):
        _do_chunk(tile_base + c * CHUNK, CHUNK, src_c, dst_c, rows, w_c)
        return 0
    lax.fori_loop(0, FULL_CHUNKS, _chunk_loop, 0)
    _do_chunk(tile_base + FULL_CHUNKS * CHUNK, TAIL, src_t, dst_t, rows_t,
              w_t)

    plsc.subcore_barrier()

    pltpu.sync_copy(num_sh.at[pl.ds(base_r, ROWS_PER_TILE)],
                    num_hbm.at[cid, pl.ds(base_r, ROWS_PER_TILE)])

    @pl.when(sid < 10)
    def _dden():
        pltpu.sync_copy(den_sh.at[pl.ds(sid * 1000, 1000)],
                        den_hbm.at[cid, pl.ds(sid * 1000, 1000)])


def _finish_body(num_ref, den_ref, b_ref, out_ref):
    s = num_ref[0] + num_ref[1]
    d = den_ref[0] + den_ref[1] + 1e-16
    out_ref[...] = jnp.maximum(s / d + b_ref[...], 0.0)


def _finish_call(num, den, b):
    return pl.pallas_call(
        _finish_body,
        out_shape=jax.ShapeDtypeStruct((N_NODES, OUT_CH), jnp.float32),
    )(num, den, b)


@jax.jit
def kernel(x, edge_index, W, att_src, att_dst, b):
    src = edge_index[0].astype(jnp.int32)
    dst = edge_index[1].astype(jnp.int32)
    h, a_s, a_d, gm = _prep_call(x, W, att_src.reshape(1, OUT_CH),
                                 att_dst.reshape(1, OUT_CH))
    gm16 = jnp.broadcast_to(gm.reshape(1), (L,))
    num, den = _sc_edges(src, dst, a_s.reshape(N_NODES), a_d.reshape(N_NODES),
                         gm16, h)
    out = _finish_call(num, den.reshape(NC, N_NODES, 1), b.reshape(1, OUT_CH))
    return out


# speedup vs baseline: 28.2486x; 28.2486x over previous
import functools

import jax
import jax.numpy as jnp
from jax import lax
from jax.experimental import pallas as pl
from jax.experimental.pallas import tpu as pltpu
from jax.experimental.pallas import tpu_sc as plsc

N_NODES = 10000
N_EDGES = 320000
OUT_CH = 128

NC = 2
NS = 16
L = 16
NW = NC * NS

E_PER_TILE = N_EDGES // NW
CHUNK = 128
FULL_CHUNKS = E_PER_TILE // CHUNK
TAIL = E_PER_TILE - FULL_CHUNKS * CHUNK
ROWS_PER_TILE = N_NODES // NS


def _prep_body(x_ref, w_ref, asrc_ref, adst_ref, h_ref, as_ref, ad_ref,
               gm_ref):
    h = jnp.dot(x_ref[...], w_ref[...], preferred_element_type=jnp.float32)
    h_ref[...] = h
    a_s = jnp.sum(h * asrc_ref[...], axis=-1, keepdims=True)
    a_d = jnp.sum(h * adst_ref[...], axis=-1, keepdims=True)
    as_ref[...] = a_s
    ad_ref[...] = a_d
    g = jnp.max(a_s) + jnp.max(a_d)
    gm_ref[0, 0] = jnp.where(g >= 0.0, g, 0.2 * g)


def _prep_call(x, W, att_src, att_dst):
    return pl.pallas_call(
        _prep_body,
        out_shape=[
            jax.ShapeDtypeStruct((N_NODES, OUT_CH), jnp.float32),
            jax.ShapeDtypeStruct((N_NODES, 1), jnp.float32),
            jax.ShapeDtypeStruct((N_NODES, 1), jnp.float32),
            jax.ShapeDtypeStruct((1, 1), jnp.float32),
        ],
        out_specs=[
            pl.BlockSpec(memory_space=pltpu.VMEM),
            pl.BlockSpec(memory_space=pltpu.VMEM),
            pl.BlockSpec(memory_space=pltpu.VMEM),
            pl.BlockSpec(memory_space=pltpu.SMEM),
        ],
    )(x, W, att_src, att_dst)


_mesh = plsc.VectorSubcoreMesh(core_axis_name="c", subcore_axis_name="s",
                               num_cores=NC, num_subcores=NS)


@functools.partial(
    pl.kernel,
    out_type=[
        jax.ShapeDtypeStruct((NC, N_NODES, OUT_CH), jnp.float32),
        jax.ShapeDtypeStruct((NC * N_NODES,), jnp.float32),
    ],
    mesh=_mesh,
    compiler_params=pltpu.CompilerParams(needs_layout_passes=False),
    scratch_types=[
        pltpu.VMEM((N_NODES,), jnp.float32),
        pltpu.VMEM((N_NODES,), jnp.float32),
        pltpu.VMEM((L,), jnp.float32),
        pltpu.VMEM((CHUNK,), jnp.int32),
        pltpu.VMEM((CHUNK,), jnp.int32),
        pltpu.VMEM((CHUNK, OUT_CH), jnp.float32),
        pltpu.VMEM((CHUNK,), jnp.float32),
        pltpu.VMEM((TAIL,), jnp.int32),
        pltpu.VMEM((TAIL,), jnp.int32),
        pltpu.VMEM((TAIL, OUT_CH), jnp.float32),
        pltpu.VMEM((TAIL,), jnp.float32),
        pltpu.VMEM((1024,), jnp.float32),
        pltpu.VMEM_SHARED((N_NODES, OUT_CH), jnp.float32),
        pltpu.VMEM_SHARED((N_NODES,), jnp.float32),
        pltpu.SemaphoreType.DMA,
    ],
)
def _sc_edges(src_hbm, dst_hbm, as_hbm, ad_hbm, gm_hbm, h_hbm,
              num_hbm, den_hbm,
              as_l, ad_l, gm_l, src_c, dst_c, rows, w_c,
              src_t, dst_t, rows_t, w_t, zb1,
              num_sh, den_sh, sem):
    cid = lax.axis_index("c")
    sid = lax.axis_index("s")
    wid = cid * NS + sid

    pltpu.sync_copy(as_hbm, as_l)
    pltpu.sync_copy(ad_hbm, ad_l)
    pltpu.sync_copy(gm_hbm, gm_l)
    gm_vec = gm_l[...]

    zero16 = jnp.zeros((L,), jnp.float32)

    def _z1(i, _):
        zb1[pl.ds(i * L, L)] = zero16
        return 0
    lax.fori_loop(0, 1024 // L, _z1, 0)

    def _zrow(i, _):
        for j in range(OUT_CH // L):
            rows[i, pl.ds(j * L, L)] = zero16
        return 0
    lax.fori_loop(0, CHUNK, _zrow, 0)

    zbase = sid * 624
    for off in (0, 128, 256, 384, 512):
        pltpu.sync_copy(rows.at[pl.ds(0, 128)],
                        num_sh.at[pl.ds(zbase + off, 128)])

    @pl.when(sid < 10)
    def _zden():
        pltpu.sync_copy(zb1.at[pl.ds(0, 1000)],
                        den_sh.at[pl.ds(sid * 1000, 1000)])

    plsc.subcore_barrier()

    tile_base = wid * E_PER_TILE

    def _do_chunk(base, k, s_ref, d_ref, r_ref, wref):
        pltpu.sync_copy(src_hbm.at[pl.ds(base, k)], s_ref)
        pltpu.sync_copy(dst_hbm.at[pl.ds(base, k)], d_ref)
        pltpu.async_copy(h_hbm.at[s_ref], r_ref, sem).wait()
        for v in range(k // L):
            si = s_ref[pl.ds(v * L, L)]
            di = d_ref[pl.ds(v * L, L)]
            a_s = plsc.load_gather(as_l, [si])
            a_d = plsc.load_gather(ad_l, [di])
            e = a_s + a_d
            e = jnp.where(e >= 0.0, e, 0.2 * e)
            wref[pl.ds(v * L, L)] = jnp.exp(e - gm_vec)

        def _scale(v, _):
            wv = wref[pl.ds(v * L, L)]
            for jj in range(L):
                ws = wv[jj]
                row = v * L + jj
                for j in range(OUT_CH // L):
                    r_ref[row, pl.ds(j * L, L)] = (
                        r_ref[row, pl.ds(j * L, L)] * ws)
            return 0
        lax.fori_loop(0, k // L, _scale, 0)
        pltpu.sync_copy(r_ref, num_sh.at[d_ref], add=True)
        pltpu.sync_copy(wref, den_sh.at[d_ref], add=True)

    def _chunk_loop(c, _):
        _do_chunk(tile_base + c * CHUNK, CHUNK, src_c, dst_c, rows, w_c)
        return 0
    lax.fori_loop(0, FULL_CHUNKS, _chunk_loop, 0)
    _do_chunk(tile_base + FULL_CHUNKS * CHUNK, TAIL, src_t, dst_t, rows_t,
              w_t)

    plsc.subcore_barrier()

    @pl.when(sid < 15)
    def _dnum():
        pltpu.sync_copy(num_sh.at[pl.ds(sid * 632, 632)],
                        num_hbm.at[cid, pl.ds(sid * 632, 632)])

    @pl.when(sid == 15)
    def _dnum_last():
        pltpu.sync_copy(num_sh.at[pl.ds(9480, 520)],
                        num_hbm.at[cid, pl.ds(9480, 520)])

    @pl.when(sid < 10)
    def _dden():
        pltpu.sync_copy(den_sh.at[pl.ds(sid * 1000, 1000)],
                        zb1.at[pl.ds(0, 1000)])
        pltpu.sync_copy(zb1.at[pl.ds(0, 1000)],
                        den_hbm.at[pl.ds(cid * N_NODES + sid * 1000, 1000)])


def _finish_body(num_ref, den_ref, b_ref, out_ref):
    s = num_ref[0] + num_ref[1]
    d = den_ref[0] + den_ref[1] + 1e-16
    out_ref[...] = jnp.maximum(s / d + b_ref[...], 0.0)


def _finish_call(num, den, b):
    return pl.pallas_call(
        _finish_body,
        out_shape=jax.ShapeDtypeStruct((N_NODES, OUT_CH), jnp.float32),
    )(num, den, b)


@jax.jit
def kernel(x, edge_index, W, att_src, att_dst, b):
    src = edge_index[0].astype(jnp.int32)
    dst = edge_index[1].astype(jnp.int32)
    h, a_s, a_d, gm = _prep_call(x, W, att_src.reshape(1, OUT_CH),
                                 att_dst.reshape(1, OUT_CH))
    gm16 = jnp.broadcast_to(gm.reshape(1), (L,))
    num, den = _sc_edges(src, dst, a_s.reshape(N_NODES), a_d.reshape(N_NODES),
                         gm16, h)
    den = den.reshape(NC, N_NODES)
    out = _finish_call(num, den.reshape(NC, N_NODES, 1), b.reshape(1, OUT_CH))
    return out
